# baseline (speedup 1.0000x reference)
import functools

import jax
import jax.numpy as jnp
from jax import lax
from jax.experimental import pallas as pl
from jax.experimental.pallas import tpu as pltpu
from jax.experimental.pallas import tpu_sc as plsc

D_MODEL = 64
B_TOTAL = 4096 * 200
NUM_CORES = 2
NUM_SUBCORES = 16
NW = NUM_CORES * NUM_SUBCORES
BPW = B_TOTAL // NW
CHUNK = 512
NCH = BPW // CHUNK

_mesh = plsc.VectorSubcoreMesh(core_axis_name="c", subcore_axis_name="s")


@functools.partial(
    pl.kernel,
    mesh=_mesh,
    compiler_params=pltpu.CompilerParams(use_tc_tiling_on_sc=False),
    out_type=jax.ShapeDtypeStruct((B_TOTAL, D_MODEL), jnp.float32),
    scratch_types=[
        pltpu.VMEM((BPW,), jnp.int32),
        pltpu.VMEM((CHUNK, D_MODEL), jnp.float32),
        pltpu.VMEM((CHUNK, D_MODEL), jnp.float32),
        pltpu.SemaphoreType.DMA,
        pltpu.SemaphoreType.DMA,
    ],
)
def _embed_gather(x_hbm, table_hbm, out_hbm, idx_v, rows0, rows1, sem0, sem1):
    wid = lax.axis_index("s") * NUM_CORES + lax.axis_index("c")
    base = wid * BPW

    pltpu.sync_copy(x_hbm.at[pl.ds(base, BPW)], idx_v)

    pltpu.async_copy(table_hbm.at[idx_v.at[pl.ds(0, CHUNK)]], rows0, sem0)
    pltpu.async_copy(table_hbm.at[idx_v.at[pl.ds(CHUNK, CHUNK)]], rows1, sem1)

    def body(i, carry):
        g = i * 2
        for b, (rows, sem) in enumerate(((rows0, sem0), (rows1, sem1))):
            c = g + b
            off = pl.multiple_of(c * CHUNK, CHUNK)
            pltpu.make_async_copy(
                table_hbm.at[idx_v.at[pl.ds(off, CHUNK)]], rows, sem
            ).wait()
            pltpu.sync_copy(rows, out_hbm.at[pl.ds(base + off, CHUNK)])

            @pl.when(c + 2 < NCH)
            def _issue():
                off2 = pl.multiple_of((c + 2) * CHUNK, CHUNK)
                pltpu.async_copy(
                    table_hbm.at[idx_v.at[pl.ds(off2, CHUNK)]], rows, sem
                )

        return carry

    lax.fori_loop(0, NCH // 2, body, 0)


def kernel(x, table):
    b, s = x.shape
    x_flat = x.reshape(-1).astype(jnp.int32)
    out = _embed_gather(x_flat, table)
    return out.reshape(b, s, D_MODEL)
